# baseline (speedup 1.0000x reference)
import jax
import jax.numpy as jnp
from jax import lax
from jax.experimental import pallas as pl
from jax.experimental.pallas import tpu as pltpu
from jax.experimental.pallas import tpu_sc as plsc

BATCH = 4096
SEQ = 200
DIM = 64

_B = BATCH * SEQ
_NW = 32
_BPW = _B // _NW
_NBUF = 4
_CHUNK = 400
_NCHUNK = _BPW // _CHUNK


def _embed_kernel(idx_hbm, table_hbm, out_hbm,
                  idx_v, b0, b1, b2, b3,
                  g0, g1, g2, g3, w0, w1, w2, w3):
    nc = 2
    wid = lax.axis_index("s") * nc + lax.axis_index("c")
    base = wid * _BPW
    pltpu.sync_copy(idx_hbm.at[pl.ds(base, _BPW)], idx_v)

    bufs = (b0, b1, b2, b3)
    gsems = (g0, g1, g2, g3)
    wsems = (w0, w1, w2, w3)

    def gather_start(g, b):
        pltpu.async_copy(
            table_hbm.at[idx_v.at[pl.ds(g * _CHUNK, _CHUNK)]], bufs[b],
            gsems[b])

    def gather_wait(b):
        pltpu.make_async_copy(
            out_hbm.at[pl.ds(base, _CHUNK)], bufs[b], gsems[b]).wait()

    def write_start(g, b):
        pltpu.async_copy(
            bufs[b], out_hbm.at[pl.ds(base + g * _CHUNK, _CHUNK)], wsems[b])

    def write_wait(b):
        pltpu.make_async_copy(
            bufs[b], out_hbm.at[pl.ds(base, _CHUNK)], wsems[b]).wait()

    for g in range(_NBUF - 1):
        gather_start(g, g)

    def body(i, carry):
        del carry
        for par in range(_NBUF):
            gg = i * _NBUF + par
            gather_wait(par)
            write_start(gg, par)
            nxt = (par + _NBUF - 1) % _NBUF
            @pl.when(jnp.logical_and(gg >= 1, gg + _NBUF - 1 < _NCHUNK))
            def _():
                write_wait(nxt)
            @pl.when(gg + _NBUF - 1 < _NCHUNK)
            def _():
                gather_start(gg + _NBUF - 1, nxt)
        return 0

    lax.fori_loop(0, _NCHUNK // _NBUF, body, 0)
    for b in range(_NBUF):
        write_wait(b)


@jax.jit
def kernel(input_ids, embed_tokens):
    idx = input_ids.reshape(-1).astype(jnp.int32)
    mesh = plsc.VectorSubcoreMesh(core_axis_name="c", subcore_axis_name="s")
    out = pl.kernel(
        _embed_kernel,
        mesh=mesh,
        compiler_params=pltpu.CompilerParams(use_tc_tiling_on_sc=False),
        out_type=jax.ShapeDtypeStruct((_B, DIM), jnp.float32),
        scratch_types=[
            pltpu.VMEM((_BPW,), jnp.int32),
            pltpu.VMEM((_CHUNK, DIM), jnp.float32),
            pltpu.VMEM((_CHUNK, DIM), jnp.float32),
            pltpu.VMEM((_CHUNK, DIM), jnp.float32),
            pltpu.VMEM((_CHUNK, DIM), jnp.float32),
            pltpu.SemaphoreType.DMA,
            pltpu.SemaphoreType.DMA,
            pltpu.SemaphoreType.DMA,
            pltpu.SemaphoreType.DMA,
            pltpu.SemaphoreType.DMA,
            pltpu.SemaphoreType.DMA,
            pltpu.SemaphoreType.DMA,
            pltpu.SemaphoreType.DMA,
        ],
    )(idx, embed_tokens)
    return out.reshape(BATCH, SEQ, DIM)
